# baseline (speedup 1.0000x reference)
import jax
import jax.numpy as jnp
from jax.experimental import pallas as pl
from jax.experimental.pallas import tpu as pltpu

_BASIS = 256
_R_DENOM = 65535.0
_INT8_DENOM = 127.0
_F8 = jnp.float8_e4m3fn


def _fused_kernel(x_ref, rqf8_ref, basis_ref, codes_ref, scales_ref, bias_ref,
                  o_ref, xf8_s, p_s, g_s):
    i = pl.program_id(0)
    j = pl.program_id(1)

    @pl.when(j == 0)
    def _():
        xb = x_ref[...].astype(jnp.bfloat16)
        xf8_s[...] = xb.astype(_F8)
        p = jax.lax.dot_general(
            xb, basis_ref[...],
            dimension_numbers=(((1,), (1,)), ((), ())),
            preferred_element_type=jnp.float32)
        p_s[...] = p.astype(jnp.bfloat16)

    @pl.when(i == 0)
    def _():
        c = codes_ref[...]
        idx = c & 0xFF
        r = ((c >> 8) & 0xFFFF).astype(jnp.float32) * (1.0 / _R_DENOM)
        bn = c.shape[1]
        row = jax.lax.broadcasted_iota(jnp.int32, (_BASIS, bn), 0)
        g_s[j] = jnp.where(row == idx, r, 0.0).astype(jnp.bfloat16)

    base = jax.lax.dot_general(
        p_s[...], g_s[j],
        dimension_numbers=(((1,), (0,)), ((), ())),
        preferred_element_type=jnp.float32)

    acc = jax.lax.dot_general(
        xf8_s[...], rqf8_ref[...],
        dimension_numbers=(((1,), (1,)), ((), ())),
        preferred_element_type=jnp.float32)

    scale = scales_ref[...] * (1.0 / _INT8_DENOM)
    o_ref[...] = acc * scale + base + bias_ref[...]


def kernel(x, codes, basis_table, residual_q, residual_scales, bias):
    b, s, d_in = x.shape
    d_out = codes.shape[0]
    m = b * s

    bm = 1024 if m % 1024 == 0 else m
    bn = 1024 if d_out % 1024 == 0 else d_out

    x2d = x.reshape(m, d_in)
    rqf8 = (residual_q - 128).astype(jnp.float32).astype(_F8)
    basis_bf = basis_table.astype(jnp.bfloat16)
    codes_row = codes.reshape(1, d_out)
    scales_row = residual_scales.reshape(1, d_out)
    bias_row = bias.reshape(1, d_out)

    grid = (m // bm, d_out // bn)

    y2d = pl.pallas_call(
        _fused_kernel,
        grid=grid,
        in_specs=[
            pl.BlockSpec((bm, d_in), lambda i, j: (i, 0)),
            pl.BlockSpec((bn, d_in), lambda i, j: (j, 0)),
            pl.BlockSpec((_BASIS, d_in), lambda i, j: (0, 0)),
            pl.BlockSpec((1, bn), lambda i, j: (0, j)),
            pl.BlockSpec((1, bn), lambda i, j: (0, j)),
            pl.BlockSpec((1, bn), lambda i, j: (0, j)),
        ],
        out_specs=pl.BlockSpec((bm, bn), lambda i, j: (i, j)),
        out_shape=jax.ShapeDtypeStruct((m, d_out), jnp.float32),
        scratch_shapes=[
            pltpu.VMEM((bm, d_in), _F8),
            pltpu.VMEM((bm, _BASIS), jnp.bfloat16),
            pltpu.VMEM((d_out // bn, _BASIS, bn), jnp.bfloat16),
        ],
        compiler_params=pltpu.CompilerParams(
            dimension_semantics=("arbitrary", "arbitrary"),
            vmem_limit_bytes=62 * 1024 * 1024,
        ),
        name="bitfield_linear_fused",
    )(x2d, rqf8, basis_bf, codes_row, scales_row, bias_row)

    return y2d.reshape(b, s, d_out)
